# baseline (speedup 1.0000x reference)
import functools

import jax
import jax.numpy as jnp
from jax import lax
from jax.experimental import pallas as pl
from jax.experimental.pallas import tpu as pltpu
from jax.experimental.pallas import tpu_sc as plsc

_NC, _NS = 2, 16
_NW = _NC * _NS
_S = 16
_UNROLL = 32


def _sc_posenc(B, T, D):
    pos_per_w = T // _NW
    n_chunks = pos_per_w // _S
    row = _S * D
    mesh = plsc.VectorSubcoreMesh(core_axis_name="c", subcore_axis_name="s")

    @functools.partial(
        pl.kernel,
        out_type=jax.ShapeDtypeStruct((B * T * D,), jnp.float32),
        mesh=mesh,
        scratch_types=[
            pltpu.VMEM((2, row), jnp.float32),
            pltpu.VMEM((4, row), jnp.float32),
            pltpu.SemaphoreType.DMA,
            pltpu.SemaphoreType.DMA,
            pltpu.SemaphoreType.DMA,
            pltpu.SemaphoreType.DMA,
            pltpu.SemaphoreType.DMA,
            pltpu.SemaphoreType.DMA,
            pltpu.SemaphoreType.DMA,
            pltpu.SemaphoreType.DMA,
            pltpu.SemaphoreType.DMA,
            pltpu.SemaphoreType.DMA,
        ],
    )
    def body(x_hbm, emb_hbm, out_hbm, ebuf, xbuf,
             es0, es1, xs0, xs1, xs2, xs3, os0, os1, os2, os3):
        esems = (es0, es1)
        xsems = (xs0, xs1, xs2, xs3)
        osems = (os0, os1, os2, os3)
        wid = lax.axis_index("s") * _NC + lax.axis_index("c")
        e0 = wid * pos_per_w * D

        def eload(g, bank):
            return pltpu.make_async_copy(
                emb_hbm.at[pl.ds(e0 + g * row, row)], ebuf.at[bank], esems[bank])

        def xload(g, b):
            return pltpu.make_async_copy(
                x_hbm.at[pl.ds(b * T * D + e0 + g * row, row)], xbuf.at[b],
                xsems[b])

        def ostore(g, b):
            return pltpu.make_async_copy(
                xbuf.at[b], out_hbm.at[pl.ds(b * T * D + e0 + g * row, row)],
                osems[b])

        eload(0, 0).start()
        eload(1, 1).start()
        xload(0, 0).start()
        xload(0, 1).start()

        def chunk_body(g, bank):
            eload(g, bank).wait()
            for b in range(4):
                if b < 2:
                    bv = b + 2

                    @pl.when(g >= 1)
                    def _():
                        ostore(g - 1, bv).wait()

                    xload(g, bv).start()
                else:
                    bv = b - 2

                    @pl.when(g + 1 <= n_chunks - 1)
                    def _():
                        ostore(g, bv).wait()
                        xload(g + 1, bv).start()

                xload(g, b).wait()

                def add_body(i, _):
                    off = i * (_UNROLL * 16)
                    for j in range(_UNROLL):
                        v = ebuf[bank, pl.ds(off + j * 16, 16)]
                        plsc.addupdate(xbuf.at[b, pl.ds(off + j * 16, 16)], v)
                    return _

                lax.fori_loop(0, row // (_UNROLL * 16), add_body, 0)
                ostore(g, b).start()

            @pl.when(g + 2 <= n_chunks - 1)
            def _():
                eload(g + 2, bank).start()

        def group(gg, _):
            chunk_body(gg * 2, 0)
            chunk_body(gg * 2 + 1, 1)
            return _

        lax.fori_loop(0, n_chunks // 2, group, 0)

        for b in range(4):
            ostore(n_chunks - 1, b).wait()

    return body


def kernel(x, emb):
    B, T, D = x.shape
    assert T % (_NW * _S) == 0 and D % (_UNROLL * 16) == 0
    out = _sc_posenc(B, T, D)(x.reshape(-1), emb.reshape(-1))
    return out.reshape(B, T, D)


# speedup vs baseline: 5.5879x; 5.5879x over previous
import jax
import jax.numpy as jnp
from jax.experimental import pallas as pl


def _add_body(x_ref, e_ref, o_ref):
    o_ref[...] = x_ref[...] + e_ref[...]


def kernel(x, emb):
    B, T, D = x.shape
    BT = 512
    grid = (T // BT, B)
    return pl.pallas_call(
        _add_body,
        grid=grid,
        in_specs=[
            pl.BlockSpec((1, BT, D), lambda i, b: (b, i, 0)),
            pl.BlockSpec((BT, D), lambda i, b: (i, 0)),
        ],
        out_specs=pl.BlockSpec((1, BT, D), lambda i, b: (b, i, 0)),
        out_shape=jax.ShapeDtypeStruct(x.shape, x.dtype),
    )(x, emb)
